# Initial kernel scaffold; baseline (speedup 1.0000x reference)
import jax
import jax.numpy as jnp
from jax.experimental import pallas as pl


def kernel(x, edge_index, W_src, b_src, W_dst, b_dst, attn_a, gamma, beta):
    raise NotImplementedError("write your pallas kernel here")


# speedup vs baseline: 1.0346x; 1.0346x over previous
import jax
import jax.numpy as jnp
from jax.experimental import pallas as pl
from jax.experimental.pallas import tpu as pltpu

_N = 10000
_E = 320000
_D = 128
_H = 8
_DH = 16


def _mm_body(x_ref, ws_ref, bs_ref, wd_ref, bd_ref, fs_ref, fd_ref):
    xb = x_ref[...]
    fs_ref[...] = jnp.dot(xb, ws_ref[...], preferred_element_type=jnp.float32) + bs_ref[...]
    fd_ref[...] = jnp.dot(xb, wd_ref[...], preferred_element_type=jnp.float32) + bd_ref[...]


def _linear_transforms(x, W_src, b_src, W_dst, b_dst):
    grid = 10
    rows = _N // grid
    return pl.pallas_call(
        _mm_body,
        grid=(grid,),
        in_specs=[
            pl.BlockSpec((rows, _D), lambda i: (i, 0)),
            pl.BlockSpec((_D, _D), lambda i: (0, 0)),
            pl.BlockSpec((1, _D), lambda i: (0, 0)),
            pl.BlockSpec((_D, _D), lambda i: (0, 0)),
            pl.BlockSpec((1, _D), lambda i: (0, 0)),
        ],
        out_specs=[
            pl.BlockSpec((rows, _D), lambda i: (i, 0)),
            pl.BlockSpec((rows, _D), lambda i: (i, 0)),
        ],
        out_shape=[
            jax.ShapeDtypeStruct((_N, _D), jnp.float32),
            jax.ShapeDtypeStruct((_N, _D), jnp.float32),
        ],
    )(x, W_src.astype(jnp.float32), b_src.reshape(1, _D),
      W_dst.astype(jnp.float32), b_dst.reshape(1, _D))


def _bn_body(p0_ref, p1_ref, x_ref, g_ref, b_ref, out_ref):
    t = p0_ref[...] + p1_ref[...] + x_ref[...]
    mean = jnp.mean(t, axis=0, keepdims=True)
    d = t - mean
    var = jnp.mean(d * d, axis=0, keepdims=True)
    y = d * jax.lax.rsqrt(var + 1e-5) * g_ref[...] + b_ref[...]
    out_ref[...] = jnp.maximum(y, 0.0)


def _bn_relu(p0, p1, x, gamma, beta):
    return pl.pallas_call(
        _bn_body,
        out_shape=jax.ShapeDtypeStruct((_N, _D), jnp.float32),
    )(p0, p1, x, gamma.reshape(1, _D), beta.reshape(1, _D))


def kernel(x, edge_index, W_src, b_src, W_dst, b_dst, attn_a, gamma, beta):
    src = edge_index[0]
    dst = edge_index[1]
    fs, fd = _linear_transforms(x, W_src, b_src, W_dst, b_dst)

    fs3 = fs.reshape(_N, _H, _DH)
    fd3 = fd.reshape(_N, _H, _DH)
    e = jax.nn.leaky_relu(fs3[src] + fd3[dst], negative_slope=0.2)
    logits = jnp.einsum('ehd,hd->eh', e, attn_a)
    ex = jnp.exp(logits)
    denom = jax.ops.segment_sum(ex, dst, num_segments=_N)
    alpha = ex / (denom[dst] + 1e-9)
    out = jax.ops.segment_sum(fs3[src] * alpha[..., None], dst, num_segments=_N)

    feat = _bn_relu(out.reshape(_N, _D), jnp.zeros((_N, _D), jnp.float32),
                    x, gamma, beta)
    return feat, alpha[..., None]


# speedup vs baseline: 19.6968x; 19.0377x over previous
import functools

import jax
import jax.numpy as jnp
from jax import lax
from jax.experimental import pallas as pl
from jax.experimental.pallas import tpu as pltpu
from jax.experimental.pallas import tpu_sc as plsc

_N = 10000
_E = 320000
_D = 128
_H = 8
_DH = 16

_NC = 2
_NS = 16
_NW = _NC * _NS
_EPW = _E // _NW
_CH = 80
_NCHUNK = _EPW // _CH
_ZMAIN = 624
_ZTAIL = _N - _NS * _ZMAIN


def _mm_body(x_ref, ws_ref, bs_ref, wd_ref, bd_ref, fs_ref, fd_ref):
    xb = x_ref[...]
    fs_ref[...] = jnp.dot(xb, ws_ref[...], preferred_element_type=jnp.float32) + bs_ref[...]
    fd_ref[...] = jnp.dot(xb, wd_ref[...], preferred_element_type=jnp.float32) + bd_ref[...]


def _linear_transforms(x, W_src, b_src, W_dst, b_dst):
    grid = 10
    rows = _N // grid
    return pl.pallas_call(
        _mm_body,
        grid=(grid,),
        in_specs=[
            pl.BlockSpec((rows, _D), lambda i: (i, 0)),
            pl.BlockSpec((_D, _D), lambda i: (0, 0)),
            pl.BlockSpec((1, _D), lambda i: (0, 0)),
            pl.BlockSpec((_D, _D), lambda i: (0, 0)),
            pl.BlockSpec((1, _D), lambda i: (0, 0)),
        ],
        out_specs=[
            pl.BlockSpec((rows, _D), lambda i: (i, 0)),
            pl.BlockSpec((rows, _D), lambda i: (i, 0)),
        ],
        out_shape=[
            jax.ShapeDtypeStruct((_N, _D), jnp.float32),
            jax.ShapeDtypeStruct((_N, _D), jnp.float32),
        ],
    )(x, W_src, b_src.reshape(1, _D), W_dst, b_dst.reshape(1, _D))


def _sc_gather_z(fs, fd, src, dst):
    mesh = plsc.VectorSubcoreMesh(core_axis_name="c", subcore_axis_name="s")

    @functools.partial(
        pl.kernel,
        out_type=jax.ShapeDtypeStruct((_E, _D), jnp.float32),
        mesh=mesh,
        scratch_types=[
            pltpu.VMEM((_CH,), jnp.int32),
            pltpu.VMEM((_CH,), jnp.int32),
            pltpu.VMEM((_CH, _D), jnp.float32),
            pltpu.VMEM((_CH, _D), jnp.float32),
            pltpu.SemaphoreType.DMA,
        ],
    )
    def k(fs_hbm, fd_hbm, src_hbm, dst_hbm, z_hbm,
          idx_s, idx_d, rows_s, rows_d, sem):
        c = lax.axis_index("c")
        s = lax.axis_index("s")
        base = (c * _NS + s) * _EPW

        def chunk(kk, _):
            off = base + kk * _CH
            pltpu.sync_copy(src_hbm.at[pl.ds(off, _CH)], idx_s)
            pltpu.sync_copy(dst_hbm.at[pl.ds(off, _CH)], idx_d)
            pltpu.async_copy(fs_hbm.at[idx_s], rows_s, sem).wait()
            pltpu.async_copy(fd_hbm.at[idx_d], rows_d, sem).wait()

            def edge(i, _):
                for j in range(_D // 16):
                    u = rows_s[i, pl.ds(j * 16, 16)] + rows_d[i, pl.ds(j * 16, 16)]
                    rows_s[i, pl.ds(j * 16, 16)] = (
                        jnp.maximum(u, 0.0) + 0.2 * jnp.minimum(u, 0.0))
                return 0
            lax.fori_loop(0, _CH, edge, 0)

            pltpu.sync_copy(rows_s, z_hbm.at[pl.ds(off, _CH)])
            return 0
        lax.fori_loop(0, _NCHUNK, chunk, 0)

    return k(fs, fd, src, dst)


def _ex_body(z_ref, a_ref, ex_ref):
    ex_ref[...] = jnp.exp(
        jnp.dot(z_ref[...], a_ref[...], preferred_element_type=jnp.float32))


def _tc_logits(z, A128):
    grid = 32
    rows = _E // grid
    return pl.pallas_call(
        _ex_body,
        grid=(grid,),
        in_specs=[
            pl.BlockSpec((rows, _D), lambda i: (i, 0)),
            pl.BlockSpec((_D, _D), lambda i: (0, 0)),
        ],
        out_specs=pl.BlockSpec((rows, _D), lambda i: (i, 0)),
        out_shape=jax.ShapeDtypeStruct((_E, _D), jnp.float32),
    )(z, A128)


def _sc_denom(ex, dst):
    mesh = plsc.VectorSubcoreMesh(core_axis_name="c", subcore_axis_name="s")

    @functools.partial(
        pl.kernel,
        out_type=jax.ShapeDtypeStruct((_NC, _N, _D), jnp.float32),
        mesh=mesh,
        scratch_types=[
            pltpu.VMEM((_CH,), jnp.int32),
            pltpu.VMEM((_CH, _D), jnp.float32),
            pltpu.VMEM_SHARED((_N, _D), jnp.float32),
            pltpu.SemaphoreType.DMA,
        ],
    )
    def k(ex_hbm, dst_hbm, den_hbm, idx_d, exv, den_sh, sem):
        c = lax.axis_index("c")
        s = lax.axis_index("s")
        base = (c * _NS + s) * _EPW
        zc = 48

        def zrow(i, _):
            for j in range(_D // 16):
                exv[i, pl.ds(j * 16, 16)] = jnp.zeros((16,), jnp.float32)
            return 0
        lax.fori_loop(0, zc, zrow, 0)

        def zblk(t, _):
            pltpu.sync_copy(exv.at[pl.ds(0, zc)],
                            den_sh.at[pl.ds(s * _ZMAIN + t * zc, zc)])
            return 0
        lax.fori_loop(0, _ZMAIN // zc, zblk, 0)

        @pl.when(s == 0)
        def _():
            pltpu.sync_copy(exv.at[pl.ds(0, _ZTAIL)],
                            den_sh.at[pl.ds(_NS * _ZMAIN, _ZTAIL)])
        plsc.subcore_barrier()

        def chunk(kk, _):
            off = base + kk * _CH
            pltpu.sync_copy(dst_hbm.at[pl.ds(off, _CH)], idx_d)
            pltpu.sync_copy(ex_hbm.at[pl.ds(off, _CH)], exv)
            pltpu.sync_copy(exv, den_sh.at[idx_d], add=True)
            return 0
        lax.fori_loop(0, _NCHUNK, chunk, 0)

        plsc.subcore_barrier()

        def dblk(t, _):
            r0 = s * _ZMAIN + t * zc
            pltpu.sync_copy(den_sh.at[pl.ds(r0, zc)], exv.at[pl.ds(0, zc)])
            pltpu.sync_copy(exv.at[pl.ds(0, zc)], den_hbm.at[c, pl.ds(r0, zc)])
            return 0
        lax.fori_loop(0, _ZMAIN // zc, dblk, 0)

        @pl.when(s == 0)
        def _():
            pltpu.sync_copy(den_sh.at[pl.ds(_NS * _ZMAIN, _ZTAIL)],
                            exv.at[pl.ds(0, _ZTAIL)])
            pltpu.sync_copy(exv.at[pl.ds(0, _ZTAIL)],
                            den_hbm.at[c, pl.ds(_NS * _ZMAIN, _ZTAIL)])

    return k(ex, dst)


def _den_body(d0_ref, d1_ref, out_ref):
    out_ref[...] = d0_ref[...] + d1_ref[...]


def _tc_den128(den):
    return pl.pallas_call(
        _den_body,
        out_shape=jax.ShapeDtypeStruct((_N, _D), jnp.float32),
    )(den[0], den[1])


def _sc_pass2(fs, src, dst, exh, den128):
    mesh = plsc.VectorSubcoreMesh(core_axis_name="c", subcore_axis_name="s")
    ch = 40
    nchunk = _EPW // ch

    @functools.partial(
        pl.kernel,
        out_type=(
            jax.ShapeDtypeStruct((_E * _H,), jnp.float32),
            jax.ShapeDtypeStruct((_NC, _N, _D), jnp.float32),
        ),
        mesh=mesh,
        scratch_types=[
            pltpu.VMEM((ch,), jnp.int32),
            pltpu.VMEM((ch,), jnp.int32),
            pltpu.VMEM((ch, _D), jnp.float32),
            pltpu.VMEM((ch, _D), jnp.float32),
            pltpu.VMEM((ch, _D), jnp.float32),
            pltpu.VMEM((ch * _H + 8,), jnp.float32),
            pltpu.VMEM((ch, _D), jnp.float32),
            pltpu.VMEM_SHARED((_N, _D), jnp.float32),
            pltpu.SemaphoreType.DMA,
        ],
    )
    def k(fs_hbm, src_hbm, dst_hbm, ex_hbm, den_hbm, al_hbm, out_hbm,
          idx_s, idx_d, rows_s, dbuf, exv, albuf, msg, out_sh, sem):
        c = lax.axis_index("c")
        s = lax.axis_index("s")
        base = (c * _NS + s) * _EPW
        zc = 24

        def zrow(i, _):
            for j in range(_D // 16):
                msg[i, pl.ds(j * 16, 16)] = jnp.zeros((16,), jnp.float32)
            return 0
        lax.fori_loop(0, zc, zrow, 0)

        def zblk(t, _):
            pltpu.sync_copy(msg.at[pl.ds(0, zc)],
                            out_sh.at[pl.ds(s * _ZMAIN + t * zc, zc)])
            return 0
        lax.fori_loop(0, _ZMAIN // zc, zblk, 0)

        @pl.when(s == 0)
        def _():
            pltpu.sync_copy(msg.at[pl.ds(0, _ZTAIL)],
                            out_sh.at[pl.ds(_NS * _ZMAIN, _ZTAIL)])
        plsc.subcore_barrier()

        def chunk(kk, _):
            off = base + kk * ch
            pltpu.sync_copy(src_hbm.at[pl.ds(off, ch)], idx_s)
            pltpu.sync_copy(dst_hbm.at[pl.ds(off, ch)], idx_d)
            pltpu.async_copy(fs_hbm.at[idx_s], rows_s, sem).wait()
            pltpu.async_copy(den_hbm.at[idx_d], dbuf, sem).wait()
            pltpu.sync_copy(ex_hbm.at[pl.ds(off, ch)], exv)

            def edge(i, _):
                den_v = dbuf[i, pl.ds(0, 16)] + 1e-9
                a_v = exv[i, pl.ds(0, 16)] / den_v
                albuf[pl.ds(i * _H, 16)] = a_v
                for h in range(_H):
                    msg[i, pl.ds(h * 16, 16)] = rows_s[i, pl.ds(h * 16, 16)] * a_v[h]
                return 0
            lax.fori_loop(0, ch, edge, 0)

            pltpu.sync_copy(albuf.at[pl.ds(0, ch * _H)],
                            al_hbm.at[pl.ds(off * _H, ch * _H)])
            pltpu.sync_copy(msg, out_sh.at[idx_d], add=True)
            return 0
        lax.fori_loop(0, nchunk, chunk, 0)

        plsc.subcore_barrier()

        def dblk(t, _):
            r0 = s * _ZMAIN + t * zc
            pltpu.sync_copy(out_sh.at[pl.ds(r0, zc)], msg.at[pl.ds(0, zc)])
            pltpu.sync_copy(msg.at[pl.ds(0, zc)], out_hbm.at[c, pl.ds(r0, zc)])
            return 0
        lax.fori_loop(0, _ZMAIN // zc, dblk, 0)

        @pl.when(s == 0)
        def _():
            pltpu.sync_copy(out_sh.at[pl.ds(_NS * _ZMAIN, _ZTAIL)],
                            msg.at[pl.ds(0, _ZTAIL)])
            pltpu.sync_copy(msg.at[pl.ds(0, _ZTAIL)],
                            out_hbm.at[c, pl.ds(_NS * _ZMAIN, _ZTAIL)])

    return k(fs, src, dst, exh, den128)


def _bn_body(p0_ref, p1_ref, x_ref, g_ref, b_ref, out_ref):
    t = p0_ref[...] + p1_ref[...] + x_ref[...]
    mean = jnp.mean(t, axis=0, keepdims=True)
    d = t - mean
    var = jnp.mean(d * d, axis=0, keepdims=True)
    y = d * jax.lax.rsqrt(var + 1e-5) * g_ref[...] + b_ref[...]
    out_ref[...] = jnp.maximum(y, 0.0)


def _bn_relu(p0, p1, x, gamma, beta):
    return pl.pallas_call(
        _bn_body,
        out_shape=jax.ShapeDtypeStruct((_N, _D), jnp.float32),
    )(p0, p1, x, gamma.reshape(1, _D), beta.reshape(1, _D))


def kernel(x, edge_index, W_src, b_src, W_dst, b_dst, attn_a, gamma, beta):
    src = edge_index[0]
    dst = edge_index[1]
    A128 = (attn_a[:, :, None] * jnp.eye(_H, dtype=attn_a.dtype)[:, None, :])
    A128 = jnp.pad(A128.reshape(_D, _H), ((0, 0), (0, _D - _H)))

    fs, fd = _linear_transforms(x, W_src, b_src, W_dst, b_dst)
    z = _sc_gather_z(fs, fd, src, dst)
    exh = _tc_logits(z, A128)
    den = _sc_denom(exh, dst)
    den128 = _tc_den128(den)
    al8, outp = _sc_pass2(fs, src, dst, exh, den128)
    feat = _bn_relu(outp[0], outp[1], x, gamma, beta)
    return feat, al8.reshape(_E, _H, 1)


# speedup vs baseline: 26.3139x; 1.3359x over previous
import functools

import jax
import jax.numpy as jnp
from jax import lax
from jax.experimental import pallas as pl
from jax.experimental.pallas import tpu as pltpu
from jax.experimental.pallas import tpu_sc as plsc

_N = 10000
_E = 320000
_D = 128
_H = 8
_DH = 16

_NC = 2
_NS = 16
_NW = _NC * _NS
_EPW = _E // _NW
_CH = 80
_NCHUNK = _EPW // _CH
_ZMAIN = 624
_ZTAIL = _N - _NS * _ZMAIN


def _mm_body(x_ref, ws_ref, bs_ref, wd_ref, bd_ref, fs_ref, fd_ref):
    xb = x_ref[...]
    fs_ref[...] = jnp.dot(xb, ws_ref[...], preferred_element_type=jnp.float32) + bs_ref[...]
    fd_ref[...] = jnp.dot(xb, wd_ref[...], preferred_element_type=jnp.float32) + bd_ref[...]


def _linear_transforms(x, W_src, b_src, W_dst, b_dst):
    grid = 10
    rows = _N // grid
    return pl.pallas_call(
        _mm_body,
        grid=(grid,),
        in_specs=[
            pl.BlockSpec((rows, _D), lambda i: (i, 0)),
            pl.BlockSpec((_D, _D), lambda i: (0, 0)),
            pl.BlockSpec((1, _D), lambda i: (0, 0)),
            pl.BlockSpec((_D, _D), lambda i: (0, 0)),
            pl.BlockSpec((1, _D), lambda i: (0, 0)),
        ],
        out_specs=[
            pl.BlockSpec((rows, _D), lambda i: (i, 0)),
            pl.BlockSpec((rows, _D), lambda i: (i, 0)),
        ],
        out_shape=[
            jax.ShapeDtypeStruct((_N, _D), jnp.float32),
            jax.ShapeDtypeStruct((_N, _D), jnp.float32),
        ],
    )(x, W_src, b_src.reshape(1, _D), W_dst, b_dst.reshape(1, _D))


def _sc_gather_z(fs, fd, src3, dst3):
    mesh = plsc.VectorSubcoreMesh(core_axis_name="c", subcore_axis_name="s")

    @functools.partial(
        pl.kernel,
        out_type=jax.ShapeDtypeStruct((_E, _D), jnp.float32),
        mesh=mesh,
        scratch_types=[
            pltpu.VMEM((_NCHUNK, _CH), jnp.int32),
            pltpu.VMEM((_NCHUNK, _CH), jnp.int32),
            pltpu.VMEM((_CH, _D), jnp.float32),
            pltpu.VMEM((_CH, _D), jnp.float32),
            pltpu.SemaphoreType.DMA,
        ],
    )
    def k(fs_hbm, fd_hbm, src_hbm, dst_hbm, z_hbm,
          idx_s, idx_d, rows_s, rows_d, sem):
        c = lax.axis_index("c")
        s = lax.axis_index("s")
        wid = c * _NS + s
        base = wid * _EPW
        pltpu.sync_copy(src_hbm.at[wid], idx_s)
        pltpu.sync_copy(dst_hbm.at[wid], idx_d)

        def chunk(kk, _):
            off = base + kk * _CH
            d1 = pltpu.async_copy(fs_hbm.at[idx_s.at[kk]], rows_s, sem)
            d2 = pltpu.async_copy(fd_hbm.at[idx_d.at[kk]], rows_d, sem)
            d1.wait()
            d2.wait()

            def edge(i, _):
                for j in range(_D // 16):
                    u = rows_s[i, pl.ds(j * 16, 16)] + rows_d[i, pl.ds(j * 16, 16)]
                    rows_s[i, pl.ds(j * 16, 16)] = (
                        jnp.maximum(u, 0.0) + 0.2 * jnp.minimum(u, 0.0))
                return 0
            lax.fori_loop(0, _CH, edge, 0)

            pltpu.sync_copy(rows_s, z_hbm.at[pl.ds(off, _CH)])
            return 0
        lax.fori_loop(0, _NCHUNK, chunk, 0)

    return k(fs, fd, src3, dst3)


def _ex_body(z_ref, a_ref, ex_ref):
    ex_ref[...] = jnp.exp(
        jnp.dot(z_ref[...], a_ref[...], preferred_element_type=jnp.float32))


def _tc_logits(z, A128):
    grid = 32
    rows = _E // grid
    return pl.pallas_call(
        _ex_body,
        grid=(grid,),
        in_specs=[
            pl.BlockSpec((rows, _D), lambda i: (i, 0)),
            pl.BlockSpec((_D, _D), lambda i: (0, 0)),
        ],
        out_specs=pl.BlockSpec((rows, _D), lambda i: (i, 0)),
        out_shape=jax.ShapeDtypeStruct((_E, _D), jnp.float32),
    )(z, A128)


def _sc_denom(ex, dst3):
    mesh = plsc.VectorSubcoreMesh(core_axis_name="c", subcore_axis_name="s")

    @functools.partial(
        pl.kernel,
        out_type=jax.ShapeDtypeStruct((_NC, _N, _D), jnp.float32),
        mesh=mesh,
        scratch_types=[
            pltpu.VMEM((_NCHUNK, _CH), jnp.int32),
            pltpu.VMEM((_CH, _D), jnp.float32),
            pltpu.VMEM_SHARED((_N, _D), jnp.float32),
            pltpu.SemaphoreType.DMA,
        ],
    )
    def k(ex_hbm, dst_hbm, den_hbm, idx_d, exv, den_sh, sem):
        c = lax.axis_index("c")
        s = lax.axis_index("s")
        wid = c * _NS + s
        base = wid * _EPW
        zc = 48
        pltpu.sync_copy(dst_hbm.at[wid], idx_d)

        def zrow(i, _):
            for j in range(_D // 16):
                exv[i, pl.ds(j * 16, 16)] = jnp.zeros((16,), jnp.float32)
            return 0
        lax.fori_loop(0, zc, zrow, 0)

        def zblk(t, _):
            pltpu.sync_copy(exv.at[pl.ds(0, zc)],
                            den_sh.at[pl.ds(s * _ZMAIN + t * zc, zc)])
            return 0
        lax.fori_loop(0, _ZMAIN // zc, zblk, 0)

        @pl.when(s == 0)
        def _():
            pltpu.sync_copy(exv.at[pl.ds(0, _ZTAIL)],
                            den_sh.at[pl.ds(_NS * _ZMAIN, _ZTAIL)])
        plsc.subcore_barrier()

        def chunk(kk, _):
            off = base + kk * _CH
            pltpu.sync_copy(ex_hbm.at[pl.ds(off, _CH)], exv)
            pltpu.sync_copy(exv, den_sh.at[idx_d.at[kk]], add=True)
            return 0
        lax.fori_loop(0, _NCHUNK, chunk, 0)

        plsc.subcore_barrier()

        def dblk(t, _):
            r0 = s * _ZMAIN + t * zc
            pltpu.sync_copy(den_sh.at[pl.ds(r0, zc)], exv.at[pl.ds(0, zc)])
            pltpu.sync_copy(exv.at[pl.ds(0, zc)], den_hbm.at[c, pl.ds(r0, zc)])
            return 0
        lax.fori_loop(0, _ZMAIN // zc, dblk, 0)

        @pl.when(s == 0)
        def _():
            pltpu.sync_copy(den_sh.at[pl.ds(_NS * _ZMAIN, _ZTAIL)],
                            exv.at[pl.ds(0, _ZTAIL)])
            pltpu.sync_copy(exv.at[pl.ds(0, _ZTAIL)],
                            den_hbm.at[c, pl.ds(_NS * _ZMAIN, _ZTAIL)])

    return k(ex, dst3)


def _den_body(d0_ref, d1_ref, out_ref):
    out_ref[...] = d0_ref[...] + d1_ref[...]


def _tc_den128(den):
    return pl.pallas_call(
        _den_body,
        out_shape=jax.ShapeDtypeStruct((_N, _D), jnp.float32),
    )(den[0], den[1])


def _sc_pass2(fs, src, dst, exh, den128):
    mesh = plsc.VectorSubcoreMesh(core_axis_name="c", subcore_axis_name="s")
    ch = 40
    nchunk = _EPW // ch

    @functools.partial(
        pl.kernel,
        out_type=(
            jax.ShapeDtypeStruct((_E * _H,), jnp.float32),
            jax.ShapeDtypeStruct((_NC, _N, _D), jnp.float32),
        ),
        mesh=mesh,
        scratch_types=[
            pltpu.VMEM((ch,), jnp.int32),
            pltpu.VMEM((ch,), jnp.int32),
            pltpu.VMEM((ch, _D), jnp.float32),
            pltpu.VMEM((ch, _D), jnp.float32),
            pltpu.VMEM((ch, _D), jnp.float32),
            pltpu.VMEM((ch * _H + 8,), jnp.float32),
            pltpu.VMEM((ch, _D), jnp.float32),
            pltpu.VMEM_SHARED((_N, _D), jnp.float32),
            pltpu.SemaphoreType.DMA,
        ],
    )
    def k(fs_hbm, src_hbm, dst_hbm, ex_hbm, den_hbm, al_hbm, out_hbm,
          idx_s, idx_d, rows_s, dbuf, exv, albuf, msg, out_sh, sem):
        c = lax.axis_index("c")
        s = lax.axis_index("s")
        base = (c * _NS + s) * _EPW
        zc = 24

        def zrow(i, _):
            for j in range(_D // 16):
                msg[i, pl.ds(j * 16, 16)] = jnp.zeros((16,), jnp.float32)
            return 0
        lax.fori_loop(0, zc, zrow, 0)

        def zblk(t, _):
            pltpu.sync_copy(msg.at[pl.ds(0, zc)],
                            out_sh.at[pl.ds(s * _ZMAIN + t * zc, zc)])
            return 0
        lax.fori_loop(0, _ZMAIN // zc, zblk, 0)

        @pl.when(s == 0)
        def _():
            pltpu.sync_copy(msg.at[pl.ds(0, _ZTAIL)],
                            out_sh.at[pl.ds(_NS * _ZMAIN, _ZTAIL)])
        plsc.subcore_barrier()

        def chunk(kk, _):
            off = base + kk * ch
            pltpu.sync_copy(src_hbm.at[pl.ds(off, ch)], idx_s)
            pltpu.sync_copy(dst_hbm.at[pl.ds(off, ch)], idx_d)
            d1 = pltpu.async_copy(fs_hbm.at[idx_s], rows_s, sem)
            d2 = pltpu.async_copy(den_hbm.at[idx_d], dbuf, sem)
            d3 = pltpu.async_copy(ex_hbm.at[pl.ds(off, ch)], exv, sem)
            d1.wait()
            d2.wait()
            d3.wait()

            def edge(i, _):
                den_v = dbuf[i, pl.ds(0, 16)] + 1e-9
                a_v = exv[i, pl.ds(0, 16)] / den_v
                albuf[pl.ds(i * _H, 16)] = a_v
                for h in range(_H):
                    msg[i, pl.ds(h * 16, 16)] = rows_s[i, pl.ds(h * 16, 16)] * a_v[h]
                return 0
            lax.fori_loop(0, ch, edge, 0)

            pltpu.sync_copy(albuf.at[pl.ds(0, ch * _H)],
                            al_hbm.at[pl.ds(off * _H, ch * _H)])
            pltpu.sync_copy(msg, out_sh.at[idx_d], add=True)
            return 0
        lax.fori_loop(0, nchunk, chunk, 0)

        plsc.subcore_barrier()

        def dblk(t, _):
            r0 = s * _ZMAIN + t * zc
            pltpu.sync_copy(out_sh.at[pl.ds(r0, zc)], msg.at[pl.ds(0, zc)])
            pltpu.sync_copy(msg.at[pl.ds(0, zc)], out_hbm.at[c, pl.ds(r0, zc)])
            return 0
        lax.fori_loop(0, _ZMAIN // zc, dblk, 0)

        @pl.when(s == 0)
        def _():
            pltpu.sync_copy(out_sh.at[pl.ds(_NS * _ZMAIN, _ZTAIL)],
                            msg.at[pl.ds(0, _ZTAIL)])
            pltpu.sync_copy(msg.at[pl.ds(0, _ZTAIL)],
                            out_hbm.at[c, pl.ds(_NS * _ZMAIN, _ZTAIL)])

    return k(fs, src, dst, exh, den128)


def _bn_body(p0_ref, p1_ref, x_ref, g_ref, b_ref, out_ref):
    t = p0_ref[...] + p1_ref[...] + x_ref[...]
    mean = jnp.mean(t, axis=0, keepdims=True)
    d = t - mean
    var = jnp.mean(d * d, axis=0, keepdims=True)
    y = d * jax.lax.rsqrt(var + 1e-5) * g_ref[...] + b_ref[...]
    out_ref[...] = jnp.maximum(y, 0.0)


def _bn_relu(p0, p1, x, gamma, beta):
    return pl.pallas_call(
        _bn_body,
        out_shape=jax.ShapeDtypeStruct((_N, _D), jnp.float32),
    )(p0, p1, x, gamma.reshape(1, _D), beta.reshape(1, _D))


def kernel(x, edge_index, W_src, b_src, W_dst, b_dst, attn_a, gamma, beta):
    src = edge_index[0]
    dst = edge_index[1]
    A128 = (attn_a[:, :, None] * jnp.eye(_H, dtype=attn_a.dtype)[:, None, :])
    A128 = jnp.pad(A128.reshape(_D, _H), ((0, 0), (0, _D - _H)))

    src3 = src.reshape(_NW, _NCHUNK, _CH)
    dst3 = dst.reshape(_NW, _NCHUNK, _CH)
    fs, fd = _linear_transforms(x, W_src, b_src, W_dst, b_dst)
    z = _sc_gather_z(fs, fd, src3, dst3)
    exh = _tc_logits(z, A128)
    den = _sc_denom(exh, dst3)
    den128 = _tc_den128(den)
    al8, outp = _sc_pass2(fs, src, dst, exh, den128)
    feat = _bn_relu(outp[0], outp[1], x, gamma, beta)
    return feat, al8.reshape(_E, _H, 1)


# speedup vs baseline: 32.4218x; 1.2321x over previous
import functools

import jax
import jax.numpy as jnp
from jax import lax
from jax.experimental import pallas as pl
from jax.experimental.pallas import tpu as pltpu
from jax.experimental.pallas import tpu_sc as plsc

_N = 10000
_E = 320000
_D = 128
_H = 8
_DH = 16

_NC = 2
_NS = 16
_NW = _NC * _NS
_EPW = _E // _NW
_CH = 80
_NCHUNK = _EPW // _CH
_ZMAIN = 624
_ZTAIL = _N - _NS * _ZMAIN


def _mm_body(x_ref, ws_ref, bs_ref, wd_ref, bd_ref, fs_ref, fd_ref):
    xb = x_ref[...]
    fs_ref[...] = jnp.dot(xb, ws_ref[...], preferred_element_type=jnp.float32) + bs_ref[...]
    fd_ref[...] = jnp.dot(xb, wd_ref[...], preferred_element_type=jnp.float32) + bd_ref[...]


def _linear_transforms(x, W_src, b_src, W_dst, b_dst):
    grid = 10
    rows = _N // grid
    return pl.pallas_call(
        _mm_body,
        grid=(grid,),
        in_specs=[
            pl.BlockSpec((rows, _D), lambda i: (i, 0)),
            pl.BlockSpec((_D, _D), lambda i: (0, 0)),
            pl.BlockSpec((1, _D), lambda i: (0, 0)),
            pl.BlockSpec((_D, _D), lambda i: (0, 0)),
            pl.BlockSpec((1, _D), lambda i: (0, 0)),
        ],
        out_specs=[
            pl.BlockSpec((rows, _D), lambda i: (i, 0)),
            pl.BlockSpec((rows, _D), lambda i: (i, 0)),
        ],
        out_shape=[
            jax.ShapeDtypeStruct((_N, _D), jnp.float32),
            jax.ShapeDtypeStruct((_N, _D), jnp.float32),
        ],
    )(x, W_src, b_src.reshape(1, _D), W_dst, b_dst.reshape(1, _D))


def _sc_gather_z(fs, fd, src3, dst3):
    mesh = plsc.VectorSubcoreMesh(core_axis_name="c", subcore_axis_name="s")

    @functools.partial(
        pl.kernel,
        out_type=jax.ShapeDtypeStruct((_E, _D), jnp.float32),
        mesh=mesh,
        scratch_types=[
            pltpu.VMEM((_NCHUNK, _CH), jnp.int32),
            pltpu.VMEM((_NCHUNK, _CH), jnp.int32),
            pltpu.VMEM((_CH, _D), jnp.float32),
            pltpu.VMEM((_CH, _D), jnp.float32),
            pltpu.VMEM((_CH, _D), jnp.float32),
            pltpu.VMEM((_CH, _D), jnp.float32),
            pltpu.SemaphoreType.DMA,
            pltpu.SemaphoreType.DMA,
        ],
    )
    def k(fs_hbm, fd_hbm, src_hbm, dst_hbm, z_hbm,
          idx_s, idx_d, rs0, rd0, rs1, rd1, sem0, sem1):
        c = lax.axis_index("c")
        s = lax.axis_index("s")
        wid = c * _NS + s
        base = wid * _EPW
        pltpu.sync_copy(src_hbm.at[wid], idx_s)
        pltpu.sync_copy(dst_hbm.at[wid], idx_d)

        def issue(kk, rs, rd, sem):
            pltpu.async_copy(fs_hbm.at[idx_s.at[kk]], rs, sem)
            pltpu.async_copy(fd_hbm.at[idx_d.at[kk]], rd, sem)

        def process(kk, rs, rd, sem):
            pltpu.make_async_copy(fs_hbm.at[idx_s.at[kk]], rs, sem).wait()
            pltpu.make_async_copy(fd_hbm.at[idx_d.at[kk]], rd, sem).wait()

            def edge(i, _):
                for j in range(_D // 16):
                    u = rs[i, pl.ds(j * 16, 16)] + rd[i, pl.ds(j * 16, 16)]
                    rs[i, pl.ds(j * 16, 16)] = (
                        jnp.maximum(u, 0.0) + 0.2 * jnp.minimum(u, 0.0))
                return 0
            lax.fori_loop(0, _CH, edge, 0)
            pltpu.sync_copy(rs, z_hbm.at[pl.ds(base + kk * _CH, _CH)])

        issue(0, rs0, rd0, sem0)

        def pair(t, _):
            kk = t * 2
            issue(kk + 1, rs1, rd1, sem1)
            process(kk, rs0, rd0, sem0)
            issue(kk + 2, rs0, rd0, sem0)
            process(kk + 1, rs1, rd1, sem1)
            return 0
        lax.fori_loop(0, (_NCHUNK - 1) // 2, pair, 0)
        process(_NCHUNK - 1, rs0, rd0, sem0)

    return k(fs, fd, src3, dst3)


def _ex_body(z_ref, a_ref, ex_ref):
    ex_ref[...] = jnp.exp(
        jnp.dot(z_ref[...], a_ref[...], preferred_element_type=jnp.float32))


def _tc_logits(z, A128):
    grid = 32
    rows = _E // grid
    return pl.pallas_call(
        _ex_body,
        grid=(grid,),
        in_specs=[
            pl.BlockSpec((rows, _D), lambda i: (i, 0)),
            pl.BlockSpec((_D, _D), lambda i: (0, 0)),
        ],
        out_specs=pl.BlockSpec((rows, _D), lambda i: (i, 0)),
        out_shape=jax.ShapeDtypeStruct((_E, _D), jnp.float32),
    )(z, A128)


def _sc_denom(ex, dst3):
    mesh = plsc.VectorSubcoreMesh(core_axis_name="c", subcore_axis_name="s")

    @functools.partial(
        pl.kernel,
        out_type=jax.ShapeDtypeStruct((_NC, _N, _D), jnp.float32),
        mesh=mesh,
        scratch_types=[
            pltpu.VMEM((_NCHUNK, _CH), jnp.int32),
            pltpu.VMEM((_CH, _D), jnp.float32),
            pltpu.VMEM((_CH, _D), jnp.float32),
            pltpu.VMEM_SHARED((_N, _D), jnp.float32),
            pltpu.SemaphoreType.DMA,
            pltpu.SemaphoreType.DMA,
        ],
    )
    def k(ex_hbm, dst_hbm, den_hbm, idx_d, ex0, ex1, den_sh, sem0, sem1):
        c = lax.axis_index("c")
        s = lax.axis_index("s")
        wid = c * _NS + s
        base = wid * _EPW
        zc = 48
        pltpu.sync_copy(dst_hbm.at[wid], idx_d)

        def zrow(i, _):
            for j in range(_D // 16):
                ex0[i, pl.ds(j * 16, 16)] = jnp.zeros((16,), jnp.float32)
            return 0
        lax.fori_loop(0, zc, zrow, 0)

        def zblk(t, _):
            pltpu.sync_copy(ex0.at[pl.ds(0, zc)],
                            den_sh.at[pl.ds(s * _ZMAIN + t * zc, zc)])
            return 0
        lax.fori_loop(0, _ZMAIN // zc, zblk, 0)

        @pl.when(s == 0)
        def _():
            pltpu.sync_copy(ex0.at[pl.ds(0, _ZTAIL)],
                            den_sh.at[pl.ds(_NS * _ZMAIN, _ZTAIL)])
        plsc.subcore_barrier()

        def issue(kk, exv, sem):
            pltpu.async_copy(ex_hbm.at[pl.ds(base + kk * _CH, _CH)], exv, sem)

        def process(kk, exv, sem):
            pltpu.make_async_copy(
                ex_hbm.at[pl.ds(base + kk * _CH, _CH)], exv, sem).wait()
            pltpu.sync_copy(exv, den_sh.at[idx_d.at[kk]], add=True)

        issue(0, ex0, sem0)

        def pair(t, _):
            kk = t * 2
            issue(kk + 1, ex1, sem1)
            process(kk, ex0, sem0)
            issue(kk + 2, ex0, sem0)
            process(kk + 1, ex1, sem1)
            return 0
        lax.fori_loop(0, (_NCHUNK - 1) // 2, pair, 0)
        process(_NCHUNK - 1, ex0, sem0)

        plsc.subcore_barrier()

        def dblk(t, _):
            r0 = s * _ZMAIN + t * zc
            pltpu.sync_copy(den_sh.at[pl.ds(r0, zc)], ex0.at[pl.ds(0, zc)])
            pltpu.sync_copy(ex0.at[pl.ds(0, zc)], den_hbm.at[c, pl.ds(r0, zc)])
            return 0
        lax.fori_loop(0, _ZMAIN // zc, dblk, 0)

        @pl.when(s == 0)
        def _():
            pltpu.sync_copy(den_sh.at[pl.ds(_NS * _ZMAIN, _ZTAIL)],
                            ex0.at[pl.ds(0, _ZTAIL)])
            pltpu.sync_copy(ex0.at[pl.ds(0, _ZTAIL)],
                            den_hbm.at[c, pl.ds(_NS * _ZMAIN, _ZTAIL)])

    return k(ex, dst3)


def _den_body(d0_ref, d1_ref, out_ref):
    out_ref[...] = d0_ref[...] + d1_ref[...]


def _tc_den128(den):
    return pl.pallas_call(
        _den_body,
        out_shape=jax.ShapeDtypeStruct((_N, _D), jnp.float32),
    )(den[0], den[1])


def _sc_pass2(fs, src, dst, exh, den128):
    mesh = plsc.VectorSubcoreMesh(core_axis_name="c", subcore_axis_name="s")
    ch = 40
    nchunk = _EPW // ch

    @functools.partial(
        pl.kernel,
        out_type=(
            jax.ShapeDtypeStruct((_E * _H,), jnp.float32),
            jax.ShapeDtypeStruct((_NC, _N, _D), jnp.float32),
        ),
        mesh=mesh,
        scratch_types=[
            pltpu.VMEM((ch,), jnp.int32),
            pltpu.VMEM((ch,), jnp.int32),
            pltpu.VMEM((ch,), jnp.int32),
            pltpu.VMEM((ch,), jnp.int32),
            pltpu.VMEM((ch, _D), jnp.float32),
            pltpu.VMEM((ch, _D), jnp.float32),
            pltpu.VMEM((ch, _D), jnp.float32),
            pltpu.VMEM((ch, _D), jnp.float32),
            pltpu.VMEM((ch, _D), jnp.float32),
            pltpu.VMEM((ch * _H + 8,), jnp.float32),
            pltpu.VMEM((ch, _D), jnp.float32),
            pltpu.VMEM_SHARED((_N, _D), jnp.float32),
            pltpu.SemaphoreType.DMA,
            pltpu.SemaphoreType.DMA,
        ],
    )
    def k(fs_hbm, src_hbm, dst_hbm, ex_hbm, den_hbm, al_hbm, out_hbm,
          is0, id0, is1, id1, rs0, rs1, db0, db1, exv, albuf, msg,
          out_sh, sem0, sem1):
        c = lax.axis_index("c")
        s = lax.axis_index("s")
        base = (c * _NS + s) * _EPW
        zc = 24

        def zrow(i, _):
            for j in range(_D // 16):
                msg[i, pl.ds(j * 16, 16)] = jnp.zeros((16,), jnp.float32)
            return 0
        lax.fori_loop(0, zc, zrow, 0)

        def zblk(t, _):
            pltpu.sync_copy(msg.at[pl.ds(0, zc)],
                            out_sh.at[pl.ds(s * _ZMAIN + t * zc, zc)])
            return 0
        lax.fori_loop(0, _ZMAIN // zc, zblk, 0)

        @pl.when(s == 0)
        def _():
            pltpu.sync_copy(msg.at[pl.ds(0, _ZTAIL)],
                            out_sh.at[pl.ds(_NS * _ZMAIN, _ZTAIL)])
        plsc.subcore_barrier()

        def issue(kk, isx, idx, rs, db, sem):
            off = base + kk * ch
            pltpu.sync_copy(src_hbm.at[pl.ds(off, ch)], isx)
            pltpu.sync_copy(dst_hbm.at[pl.ds(off, ch)], idx)
            pltpu.async_copy(fs_hbm.at[isx], rs, sem)
            pltpu.async_copy(den_hbm.at[idx], db, sem)

        def process(kk, isx, idx, rs, db, sem):
            off = base + kk * ch
            d3 = pltpu.async_copy(ex_hbm.at[pl.ds(off, ch)], exv, sem)
            pltpu.make_async_copy(fs_hbm.at[isx], rs, sem).wait()
            pltpu.make_async_copy(den_hbm.at[idx], db, sem).wait()
            d3.wait()

            def edge(i, _):
                den_v = db[i, pl.ds(0, 16)] + 1e-9
                a_v = exv[i, pl.ds(0, 16)] / den_v
                albuf[pl.ds(i * _H, 16)] = a_v
                for h in range(_H):
                    msg[i, pl.ds(h * 16, 16)] = rs[i, pl.ds(h * 16, 16)] * a_v[h]
                return 0
            lax.fori_loop(0, ch, edge, 0)

            pltpu.sync_copy(albuf.at[pl.ds(0, ch * _H)],
                            al_hbm.at[pl.ds(off * _H, ch * _H)])
            pltpu.sync_copy(msg, out_sh.at[idx], add=True)

        issue(0, is0, id0, rs0, db0, sem0)

        def pair(t, _):
            kk = t * 2
            issue(kk + 1, is1, id1, rs1, db1, sem1)
            process(kk, is0, id0, rs0, db0, sem0)

            @pl.when(kk + 2 < nchunk)
            def _():
                issue(kk + 2, is0, id0, rs0, db0, sem0)
            process(kk + 1, is1, id1, rs1, db1, sem1)
            return 0
        lax.fori_loop(0, nchunk // 2, pair, 0)

        plsc.subcore_barrier()

        def dblk(t, _):
            r0 = s * _ZMAIN + t * zc
            pltpu.sync_copy(out_sh.at[pl.ds(r0, zc)], msg.at[pl.ds(0, zc)])
            pltpu.sync_copy(msg.at[pl.ds(0, zc)], out_hbm.at[c, pl.ds(r0, zc)])
            return 0
        lax.fori_loop(0, _ZMAIN // zc, dblk, 0)

        @pl.when(s == 0)
        def _():
            pltpu.sync_copy(out_sh.at[pl.ds(_NS * _ZMAIN, _ZTAIL)],
                            msg.at[pl.ds(0, _ZTAIL)])
            pltpu.sync_copy(msg.at[pl.ds(0, _ZTAIL)],
                            out_hbm.at[c, pl.ds(_NS * _ZMAIN, _ZTAIL)])

    return k(fs, src, dst, exh, den128)


def _bn_body(p0_ref, p1_ref, x_ref, g_ref, b_ref, out_ref):
    t = p0_ref[...] + p1_ref[...] + x_ref[...]
    mean = jnp.mean(t, axis=0, keepdims=True)
    d = t - mean
    var = jnp.mean(d * d, axis=0, keepdims=True)
    y = d * jax.lax.rsqrt(var + 1e-5) * g_ref[...] + b_ref[...]
    out_ref[...] = jnp.maximum(y, 0.0)


def _bn_relu(p0, p1, x, gamma, beta):
    return pl.pallas_call(
        _bn_body,
        out_shape=jax.ShapeDtypeStruct((_N, _D), jnp.float32),
    )(p0, p1, x, gamma.reshape(1, _D), beta.reshape(1, _D))


def kernel(x, edge_index, W_src, b_src, W_dst, b_dst, attn_a, gamma, beta):
    src = edge_index[0]
    dst = edge_index[1]
    A128 = (attn_a[:, :, None] * jnp.eye(_H, dtype=attn_a.dtype)[:, None, :])
    A128 = jnp.pad(A128.reshape(_D, _H), ((0, 0), (0, _D - _H)))

    src3 = src.reshape(_NW, _NCHUNK, _CH)
    dst3 = dst.reshape(_NW, _NCHUNK, _CH)
    fs, fd = _linear_transforms(x, W_src, b_src, W_dst, b_dst)
    z = _sc_gather_z(fs, fd, src3, dst3)
    exh = _tc_logits(z, A128)
    den = _sc_denom(exh, dst3)
    den128 = _tc_den128(den)
    al8, outp = _sc_pass2(fs, src, dst, exh, den128)
    feat = _bn_relu(outp[0], outp[1], x, gamma, beta)
    return feat, al8.reshape(_E, _H, 1)


# speedup vs baseline: 32.5432x; 1.0037x over previous
import functools

import jax
import jax.numpy as jnp
from jax import lax
from jax.experimental import pallas as pl
from jax.experimental.pallas import tpu as pltpu
from jax.experimental.pallas import tpu_sc as plsc

_N = 10000
_E = 320000
_D = 128
_H = 8
_DH = 16

_NC = 2
_NS = 16
_NW = _NC * _NS
_EPW = _E // _NW
_CH = 80
_NCHUNK = _EPW // _CH
_ZMAIN = 624
_ZTAIL = _N - _NS * _ZMAIN


def _mm_body(x_ref, ws_ref, bs_ref, wd_ref, bd_ref, fs_ref, fd_ref):
    xb = x_ref[...]
    fs_ref[...] = jnp.dot(xb, ws_ref[...], preferred_element_type=jnp.float32) + bs_ref[...]
    fd_ref[...] = jnp.dot(xb, wd_ref[...], preferred_element_type=jnp.float32) + bd_ref[...]


def _linear_transforms(x, W_src, b_src, W_dst, b_dst):
    grid = 10
    rows = _N // grid
    return pl.pallas_call(
        _mm_body,
        grid=(grid,),
        in_specs=[
            pl.BlockSpec((rows, _D), lambda i: (i, 0)),
            pl.BlockSpec((_D, _D), lambda i: (0, 0)),
            pl.BlockSpec((1, _D), lambda i: (0, 0)),
            pl.BlockSpec((_D, _D), lambda i: (0, 0)),
            pl.BlockSpec((1, _D), lambda i: (0, 0)),
        ],
        out_specs=[
            pl.BlockSpec((rows, _D), lambda i: (i, 0)),
            pl.BlockSpec((rows, _D), lambda i: (i, 0)),
        ],
        out_shape=[
            jax.ShapeDtypeStruct((_N, _D), jnp.float32),
            jax.ShapeDtypeStruct((_N, _D), jnp.float32),
        ],
    )(x, W_src, b_src.reshape(1, _D), W_dst, b_dst.reshape(1, _D))


def _sc_gather_z(fs, fd, src1, dst1):
    mesh = plsc.VectorSubcoreMesh(core_axis_name="c", subcore_axis_name="s")

    @functools.partial(
        pl.kernel,
        out_type=jax.ShapeDtypeStruct((_E, _D), jnp.float32),
        mesh=mesh,
        scratch_types=[
            pltpu.VMEM((_EPW,), jnp.int32),
            pltpu.VMEM((_EPW,), jnp.int32),
            pltpu.VMEM((_CH, _D), jnp.float32),
            pltpu.VMEM((_CH, _D), jnp.float32),
            pltpu.VMEM((_CH, _D), jnp.float32),
            pltpu.VMEM((_CH, _D), jnp.float32),
            pltpu.SemaphoreType.DMA,
            pltpu.SemaphoreType.DMA,
        ],
    )
    def k(fs_hbm, fd_hbm, src_hbm, dst_hbm, z_hbm,
          idx_s, idx_d, rs0, rd0, rs1, rd1, sem0, sem1):
        c = lax.axis_index("c")
        s = lax.axis_index("s")
        wid = c * _NS + s
        base = wid * _EPW
        pltpu.sync_copy(src_hbm.at[pl.ds(base, _EPW)], idx_s)
        pltpu.sync_copy(dst_hbm.at[pl.ds(base, _EPW)], idx_d)

        def issue(kk, rs, rd, sem):
            pltpu.async_copy(fs_hbm.at[idx_s.at[pl.ds(kk * _CH, _CH)]], rs, sem)
            pltpu.async_copy(fd_hbm.at[idx_d.at[pl.ds(kk * _CH, _CH)]], rd, sem)

        def process(kk, rs, rd, sem):
            pltpu.make_async_copy(
                fs_hbm.at[idx_s.at[pl.ds(kk * _CH, _CH)]], rs, sem).wait()
            pltpu.make_async_copy(
                fd_hbm.at[idx_d.at[pl.ds(kk * _CH, _CH)]], rd, sem).wait()

            def edge(i, _):
                for j in range(_D // 16):
                    u = rs[i, pl.ds(j * 16, 16)] + rd[i, pl.ds(j * 16, 16)]
                    rs[i, pl.ds(j * 16, 16)] = (
                        jnp.maximum(u, 0.0) + 0.2 * jnp.minimum(u, 0.0))
                return 0
            lax.fori_loop(0, _CH, edge, 0)
            pltpu.sync_copy(rs, z_hbm.at[pl.ds(base + kk * _CH, _CH)])

        issue(0, rs0, rd0, sem0)

        def pair(t, _):
            kk = t * 2
            issue(kk + 1, rs1, rd1, sem1)
            process(kk, rs0, rd0, sem0)
            issue(kk + 2, rs0, rd0, sem0)
            process(kk + 1, rs1, rd1, sem1)
            return 0
        lax.fori_loop(0, (_NCHUNK - 1) // 2, pair, 0)
        process(_NCHUNK - 1, rs0, rd0, sem0)

    return k(fs, fd, src1, dst1)


def _ex_body(z_ref, a_ref, ex_ref):
    ex_ref[...] = jnp.exp(
        jnp.dot(z_ref[...], a_ref[...], preferred_element_type=jnp.float32))


def _tc_logits(z, A128):
    grid = 32
    rows = _E // grid
    return pl.pallas_call(
        _ex_body,
        grid=(grid,),
        in_specs=[
            pl.BlockSpec((rows, _D), lambda i: (i, 0)),
            pl.BlockSpec((_D, _D), lambda i: (0, 0)),
        ],
        out_specs=pl.BlockSpec((rows, _D), lambda i: (i, 0)),
        out_shape=jax.ShapeDtypeStruct((_E, _D), jnp.float32),
    )(z, A128)


def _sc_denom(ex, dst3):
    mesh = plsc.VectorSubcoreMesh(core_axis_name="c", subcore_axis_name="s")

    @functools.partial(
        pl.kernel,
        out_type=jax.ShapeDtypeStruct((_NC, _N, _D), jnp.float32),
        mesh=mesh,
        scratch_types=[
            pltpu.VMEM((_NCHUNK, _CH), jnp.int32),
            pltpu.VMEM((_CH, _D), jnp.float32),
            pltpu.VMEM((_CH, _D), jnp.float32),
            pltpu.VMEM_SHARED((_N, _D), jnp.float32),
            pltpu.SemaphoreType.DMA,
            pltpu.SemaphoreType.DMA,
        ],
    )
    def k(ex_hbm, dst_hbm, den_hbm, idx_d, ex0, ex1, den_sh, sem0, sem1):
        c = lax.axis_index("c")
        s = lax.axis_index("s")
        wid = c * _NS + s
        base = wid * _EPW
        zc = 48
        pltpu.sync_copy(dst_hbm.at[wid], idx_d)

        def zrow(i, _):
            for j in range(_D // 16):
                ex0[i, pl.ds(j * 16, 16)] = jnp.zeros((16,), jnp.float32)
            return 0
        lax.fori_loop(0, zc, zrow, 0)

        def zblk(t, _):
            pltpu.sync_copy(ex0.at[pl.ds(0, zc)],
                            den_sh.at[pl.ds(s * _ZMAIN + t * zc, zc)])
            return 0
        lax.fori_loop(0, _ZMAIN // zc, zblk, 0)

        @pl.when(s == 0)
        def _():
            pltpu.sync_copy(ex0.at[pl.ds(0, _ZTAIL)],
                            den_sh.at[pl.ds(_NS * _ZMAIN, _ZTAIL)])
        plsc.subcore_barrier()

        def issue(kk, exv, sem):
            pltpu.async_copy(ex_hbm.at[pl.ds(base + kk * _CH, _CH)], exv, sem)

        def process(kk, exv, sem):
            pltpu.make_async_copy(
                ex_hbm.at[pl.ds(base + kk * _CH, _CH)], exv, sem).wait()
            pltpu.sync_copy(exv, den_sh.at[idx_d.at[kk]], add=True)

        issue(0, ex0, sem0)

        def pair(t, _):
            kk = t * 2
            issue(kk + 1, ex1, sem1)
            process(kk, ex0, sem0)
            issue(kk + 2, ex0, sem0)
            process(kk + 1, ex1, sem1)
            return 0
        lax.fori_loop(0, (_NCHUNK - 1) // 2, pair, 0)
        process(_NCHUNK - 1, ex0, sem0)

        plsc.subcore_barrier()

        def dblk(t, _):
            r0 = s * _ZMAIN + t * zc
            pltpu.sync_copy(den_sh.at[pl.ds(r0, zc)], ex0.at[pl.ds(0, zc)])
            pltpu.sync_copy(ex0.at[pl.ds(0, zc)], den_hbm.at[c, pl.ds(r0, zc)])
            return 0
        lax.fori_loop(0, _ZMAIN // zc, dblk, 0)

        @pl.when(s == 0)
        def _():
            pltpu.sync_copy(den_sh.at[pl.ds(_NS * _ZMAIN, _ZTAIL)],
                            ex0.at[pl.ds(0, _ZTAIL)])
            pltpu.sync_copy(ex0.at[pl.ds(0, _ZTAIL)],
                            den_hbm.at[c, pl.ds(_NS * _ZMAIN, _ZTAIL)])

    return k(ex, dst3)


def _den_body(d0_ref, d1_ref, out_ref):
    out_ref[...] = d0_ref[...] + d1_ref[...]


def _tc_den128(den):
    return pl.pallas_call(
        _den_body,
        out_shape=jax.ShapeDtypeStruct((_N, _D), jnp.float32),
    )(den[0], den[1])


def _sc_pass2(fs, src, dst, exh, den128):
    mesh = plsc.VectorSubcoreMesh(core_axis_name="c", subcore_axis_name="s")
    ch = 40
    nchunk = _EPW // ch

    @functools.partial(
        pl.kernel,
        out_type=(
            jax.ShapeDtypeStruct((_E * _H,), jnp.float32),
            jax.ShapeDtypeStruct((_NC, _N, _D), jnp.float32),
        ),
        mesh=mesh,
        scratch_types=[
            pltpu.VMEM((ch,), jnp.int32),
            pltpu.VMEM((ch,), jnp.int32),
            pltpu.VMEM((ch,), jnp.int32),
            pltpu.VMEM((ch,), jnp.int32),
            pltpu.VMEM((ch, _D), jnp.float32),
            pltpu.VMEM((ch, _D), jnp.float32),
            pltpu.VMEM((ch, _D), jnp.float32),
            pltpu.VMEM((ch, _D), jnp.float32),
            pltpu.VMEM((ch, _D), jnp.float32),
            pltpu.VMEM((ch * _H + 8,), jnp.float32),
            pltpu.VMEM((ch, _D), jnp.float32),
            pltpu.VMEM_SHARED((_N, _D), jnp.float32),
            pltpu.SemaphoreType.DMA,
            pltpu.SemaphoreType.DMA,
        ],
    )
    def k(fs_hbm, src_hbm, dst_hbm, ex_hbm, den_hbm, al_hbm, out_hbm,
          is0, id0, is1, id1, rs0, rs1, db0, db1, exv, albuf, msg,
          out_sh, sem0, sem1):
        c = lax.axis_index("c")
        s = lax.axis_index("s")
        base = (c * _NS + s) * _EPW
        zc = 24

        def zrow(i, _):
            for j in range(_D // 16):
                msg[i, pl.ds(j * 16, 16)] = jnp.zeros((16,), jnp.float32)
            return 0
        lax.fori_loop(0, zc, zrow, 0)

        def zblk(t, _):
            pltpu.sync_copy(msg.at[pl.ds(0, zc)],
                            out_sh.at[pl.ds(s * _ZMAIN + t * zc, zc)])
            return 0
        lax.fori_loop(0, _ZMAIN // zc, zblk, 0)

        @pl.when(s == 0)
        def _():
            pltpu.sync_copy(msg.at[pl.ds(0, _ZTAIL)],
                            out_sh.at[pl.ds(_NS * _ZMAIN, _ZTAIL)])
        plsc.subcore_barrier()

        def issue(kk, isx, idx, rs, db, sem):
            off = base + kk * ch
            pltpu.sync_copy(src_hbm.at[pl.ds(off, ch)], isx)
            pltpu.sync_copy(dst_hbm.at[pl.ds(off, ch)], idx)
            pltpu.async_copy(fs_hbm.at[isx], rs, sem)
            pltpu.async_copy(den_hbm.at[idx], db, sem)

        def process(kk, isx, idx, rs, db, sem):
            off = base + kk * ch
            d3 = pltpu.async_copy(ex_hbm.at[pl.ds(off, ch)], exv, sem)
            pltpu.make_async_copy(fs_hbm.at[isx], rs, sem).wait()
            pltpu.make_async_copy(den_hbm.at[idx], db, sem).wait()
            d3.wait()

            def edge(i, _):
                den_v = db[i, pl.ds(0, 16)] + 1e-9
                a_v = exv[i, pl.ds(0, 16)] / den_v
                albuf[pl.ds(i * _H, 16)] = a_v
                for h in range(_H):
                    msg[i, pl.ds(h * 16, 16)] = rs[i, pl.ds(h * 16, 16)] * a_v[h]
                return 0
            lax.fori_loop(0, ch, edge, 0)

            pltpu.sync_copy(albuf.at[pl.ds(0, ch * _H)],
                            al_hbm.at[pl.ds(off * _H, ch * _H)])
            pltpu.sync_copy(msg, out_sh.at[idx], add=True)

        issue(0, is0, id0, rs0, db0, sem0)

        def pair(t, _):
            kk = t * 2
            issue(kk + 1, is1, id1, rs1, db1, sem1)
            process(kk, is0, id0, rs0, db0, sem0)

            @pl.when(kk + 2 < nchunk)
            def _():
                issue(kk + 2, is0, id0, rs0, db0, sem0)
            process(kk + 1, is1, id1, rs1, db1, sem1)
            return 0
        lax.fori_loop(0, nchunk // 2, pair, 0)

        plsc.subcore_barrier()

        def dblk(t, _):
            r0 = s * _ZMAIN + t * zc
            pltpu.sync_copy(out_sh.at[pl.ds(r0, zc)], msg.at[pl.ds(0, zc)])
            pltpu.sync_copy(msg.at[pl.ds(0, zc)], out_hbm.at[c, pl.ds(r0, zc)])
            return 0
        lax.fori_loop(0, _ZMAIN // zc, dblk, 0)

        @pl.when(s == 0)
        def _():
            pltpu.sync_copy(out_sh.at[pl.ds(_NS * _ZMAIN, _ZTAIL)],
                            msg.at[pl.ds(0, _ZTAIL)])
            pltpu.sync_copy(msg.at[pl.ds(0, _ZTAIL)],
                            out_hbm.at[c, pl.ds(_NS * _ZMAIN, _ZTAIL)])

    return k(fs, src, dst, exh, den128)


def _bn_body(p0_ref, p1_ref, x_ref, g_ref, b_ref, out_ref):
    t = p0_ref[...] + p1_ref[...] + x_ref[...]
    mean = jnp.mean(t, axis=0, keepdims=True)
    d = t - mean
    var = jnp.mean(d * d, axis=0, keepdims=True)
    y = d * jax.lax.rsqrt(var + 1e-5) * g_ref[...] + b_ref[...]
    out_ref[...] = jnp.maximum(y, 0.0)


def _bn_relu(p0, p1, x, gamma, beta):
    return pl.pallas_call(
        _bn_body,
        out_shape=jax.ShapeDtypeStruct((_N, _D), jnp.float32),
    )(p0, p1, x, gamma.reshape(1, _D), beta.reshape(1, _D))


def kernel(x, edge_index, W_src, b_src, W_dst, b_dst, attn_a, gamma, beta):
    src = edge_index[0]
    dst = edge_index[1]
    A128 = (attn_a[:, :, None] * jnp.eye(_H, dtype=attn_a.dtype)[:, None, :])
    A128 = jnp.pad(A128.reshape(_D, _H), ((0, 0), (0, _D - _H)))

    dst3 = dst.reshape(_NW, _NCHUNK, _CH)
    fs, fd = _linear_transforms(x, W_src, b_src, W_dst, b_dst)
    z = _sc_gather_z(fs, fd, src, dst)
    exh = _tc_logits(z, A128)
    den = _sc_denom(exh, dst3)
    den128 = _tc_den128(den)
    al8, outp = _sc_pass2(fs, src, dst, exh, den128)
    feat = _bn_relu(outp[0], outp[1], x, gamma, beta)
    return feat, al8.reshape(_E, _H, 1)
